# Initial kernel scaffold; baseline (speedup 1.0000x reference)
import jax
import jax.numpy as jnp
from jax.experimental import pallas as pl


def kernel(timestamps, labels, seq_lens):
    raise NotImplementedError("write your pallas kernel here")


# speedup vs baseline: 22.1186x; 22.1186x over previous
import jax
import jax.numpy as jnp
from jax import lax
from jax.experimental import pallas as pl
from jax.experimental.pallas import tpu as pltpu
from jax.experimental.pallas import tpu_sc as plsc

_B = 64
_L = 2048
_C = 64
_HORIZONS = (16.0, 64.0, 256.0)
_NH = 3
_NC, _NS = 2, 16
_NW = _NC * _NS
_RPW = _B // _NW
_CP = 64
_NCHUNK = _L // _CP
_OUTW = _CP * _NH * _C
_ROWW = _L * _NH * _C


def _iota16():
    return lax.iota(jnp.int32, 16)


def _kernel_body(ts_hbm, lab_hbm, sl_hbm, out_hbm,
                 sl_v, idx_v, mg_v, arr_v, lab_v, t_v, off_v, inv_v,
                 ob0_v, ob1_v, sem_g, sem0, sem1):
    wid = lax.axis_index("s") * _NC + lax.axis_index("c")

    pltpu.sync_copy(sl_hbm, sl_v.at[pl.ds(0, _B)])
    for c in range(_B // 16):
        bids = _iota16() + (16 * c)
        seq = sl_v[pl.ds(16 * c, 16)]
        idx_v[pl.ds(16 * c, 16)] = bids * _L + seq - 1
    pltpu.async_copy(ts_hbm.at[idx_v], mg_v, sem_g).wait()
    mx = mg_v[pl.ds(0, 16)]
    for c in range(1, _B // 16):
        mx = jnp.maximum(mx, mg_v[pl.ds(16 * c, 16)])
    max_valid = mx[0]
    for j in range(1, 16):
        max_valid = jnp.maximum(max_valid, mx[j])

    def inv_body(c, _):
        pos = _iota16() + (16 * c)
        inv_v[pl.ds(16 * c, 16)] = 1.0 / (pos + 1).astype(jnp.float32)
        return 0

    lax.fori_loop(0, _L // 16, inv_body, 0)

    def do_row(r, _carry):
        b = wid * _RPW + r
        len_b = sl_v[pl.ds(b, 16)][0]

        pltpu.sync_copy(ts_hbm.at[pl.ds(b * _L, _L)], arr_v)
        pltpu.sync_copy(lab_hbm.at[pl.ds(b * _L, _L)], lab_v)

        def mask_body(c, _):
            sl16 = pl.ds(16 * c, 16)
            pos = _iota16() + (16 * c)
            arr_v[sl16] = jnp.where(pos < len_b, arr_v[sl16], max_valid)
            return 0

        lax.fori_loop(0, _L // 16, mask_body, 0)

        zero16 = jnp.zeros((16,), jnp.int32)
        t_v[pl.ds(0, 16)] = zero16
        t_v[pl.ds(16, 16)] = zero16

        def build_body(c, carry):
            c0, c1 = carry
            lv = lab_v[pl.ds(c * 16, 16)]
            for j in range(16):
                lbl = lv[j]
                w = jnp.bitwise_and(lbl, 31)
                incval = jnp.where(lbl >= 32, 65536, 1)
                c0 = c0 + jnp.where(_iota16() == w, incval, 0)
                c1 = c1 + jnp.where(_iota16() == w - 16, incval, 0)
                k32 = (c * 16 + j + 1) * 32
                t_v[pl.ds(k32, 16)] = c0
                t_v[pl.ds(k32 + 16, 16)] = c1
            return (c0, c1)

        lax.fori_loop(0, _L // 16, build_body, (zero16, zero16))

        arr0 = arr_v[pl.ds(0, 16)][0]

        def search_body(c, fo_carry):
            base = c * 16
            q = arr_v[pl.ds(base, 16)]
            gm = jnp.maximum(_iota16() + (base - 1), 0)
            vm1 = plsc.load_gather(arr_v, [gm])
            cand = jnp.where(q != vm1, _iota16() + base, 0)
            fo = jnp.maximum(plsc.cummax(cand), fo_carry)
            e_m1 = jnp.where(q == arr0, -1, fo)
            for hidx in range(_NH):
                qh = q - _HORIZONS[hidx]
                lo = jnp.zeros((16,), jnp.int32)
                hi = jnp.full((16,), _L + 1, jnp.int32)
                for _step in range(12):
                    mid = jnp.right_shift(lo + hi, 1)
                    g = jnp.minimum(jnp.maximum(mid - 1, 0), _L - 1)
                    v = plsc.load_gather(arr_v, [g])
                    pred = v < qh
                    lo = jnp.where(pred, mid + 1, lo)
                    hi = jnp.where(pred, hi, mid)
                s = jnp.maximum(lo - 1, 0)
                e = jnp.maximum(s, e_m1)
                off_v[pl.ds((2 * hidx) * _L + base, 16)] = e * 32
                off_v[pl.ds((2 * hidx + 1) * _L + base, 16)] = s * 32
            return fo[15]

        lax.fori_loop(0, _L // 16, search_body, jnp.int32(0))

        row_base = b * _ROWW

        def make_chunk(buf_ref, sem):
            def chunk(ch, acc):
                @pl.when(ch >= 2)
                def _wait_prev():
                    pltpu.make_async_copy(
                        buf_ref, out_hbm.at[pl.ds(0, _OUTW)], sem).wait()

                def group_body(g, acc):
                    pbase = ch * _CP + g * 16
                    iv_vec = inv_v[pl.ds(pbase, 16)]
                    eovs = [off_v[pl.ds((2 * h) * _L + pbase, 16)]
                            for h in range(_NH)]
                    sovs = [off_v[pl.ds((2 * h + 1) * _L + pbase, 16)]
                            for h in range(_NH)]
                    new_acc = list(acc)
                    for j in range(16):
                        iv = iv_vec[j]
                        obase = (g * 16 + j) * (_NH * _C)
                        for hidx in range(_NH):
                            eo = eovs[hidx][j]
                            so = sovs[hidx][j]
                            d0 = t_v[pl.ds(eo, 16)] - t_v[pl.ds(so, 16)]
                            d1 = (t_v[pl.ds(eo + 16, 16)]
                                  - t_v[pl.ds(so + 16, 16)])
                            a0 = new_acc[4 * hidx + 0] + jnp.bitwise_and(d0, 65535)
                            a1 = new_acc[4 * hidx + 1] + jnp.bitwise_and(d1, 65535)
                            a2 = new_acc[4 * hidx + 2] + jnp.right_shift(d0, 16)
                            a3 = new_acc[4 * hidx + 3] + jnp.right_shift(d1, 16)
                            new_acc[4 * hidx + 0] = a0
                            new_acc[4 * hidx + 1] = a1
                            new_acc[4 * hidx + 2] = a2
                            new_acc[4 * hidx + 3] = a3
                            ob = obase + hidx * _C
                            buf_ref[pl.ds(ob, 16)] = a0.astype(jnp.float32) * iv
                            buf_ref[pl.ds(ob + 16, 16)] = a1.astype(jnp.float32) * iv
                            buf_ref[pl.ds(ob + 32, 16)] = a2.astype(jnp.float32) * iv
                            buf_ref[pl.ds(ob + 48, 16)] = a3.astype(jnp.float32) * iv
                    return tuple(new_acc)

                acc = lax.fori_loop(0, _CP // 16, group_body, acc)
                pltpu.async_copy(
                    buf_ref, out_hbm.at[pl.ds(row_base + ch * _OUTW, _OUTW)],
                    sem)
                return acc
            return chunk

        chunk0 = make_chunk(ob0_v, sem0)
        chunk1 = make_chunk(ob1_v, sem1)

        def pair_body(j, acc):
            acc = chunk0(2 * j, acc)
            acc = chunk1(2 * j + 1, acc)
            return acc

        acc0 = tuple(jnp.zeros((16,), jnp.int32) for _ in range(4 * _NH))
        lax.fori_loop(0, _NCHUNK // 2, pair_body, acc0)

        pltpu.make_async_copy(ob0_v, out_hbm.at[pl.ds(0, _OUTW)], sem0).wait()
        pltpu.make_async_copy(ob1_v, out_hbm.at[pl.ds(0, _OUTW)], sem1).wait()
        return 0

    lax.fori_loop(0, _RPW, do_row, 0)


@jax.jit
def _run(ts_flat, lab_flat, seq_lens):
    mesh = plsc.VectorSubcoreMesh(
        core_axis_name="c", subcore_axis_name="s",
        num_cores=_NC, num_subcores=_NS)
    f = pl.kernel(
        _kernel_body,
        out_type=jax.ShapeDtypeStruct((_B * _ROWW,), jnp.float32),
        mesh=mesh,
        compiler_params=pltpu.CompilerParams(needs_layout_passes=False),
        scratch_types=[
            pltpu.VMEM((_B + 16,), jnp.int32),
            pltpu.VMEM((_B,), jnp.int32),
            pltpu.VMEM((_B,), jnp.float32),
            pltpu.VMEM((_L,), jnp.float32),
            pltpu.VMEM((_L,), jnp.int32),
            pltpu.VMEM((_L * 32 + 32,), jnp.int32),
            pltpu.VMEM((2 * _NH * _L,), jnp.int32),
            pltpu.VMEM((_L,), jnp.float32),
            pltpu.VMEM((_OUTW,), jnp.float32),
            pltpu.VMEM((_OUTW,), jnp.float32),
            pltpu.SemaphoreType.DMA,
            pltpu.SemaphoreType.DMA,
            pltpu.SemaphoreType.DMA,
        ],
    )
    return f(ts_flat, lab_flat, seq_lens)


def kernel(timestamps, labels, seq_lens):
    ts_flat = timestamps.reshape(-1)
    lab_flat = labels.reshape(-1)
    payload = _run(ts_flat, lab_flat, seq_lens).reshape(_B, _L, _NH * _C)
    return (payload, payload[None])
